# Initial kernel scaffold; baseline (speedup 1.0000x reference)
import jax
import jax.numpy as jnp
from jax.experimental import pallas as pl


def kernel(emb, labels):
    raise NotImplementedError("write your pallas kernel here")


# speedup vs baseline: 9.0157x; 9.0157x over previous
import jax
import jax.numpy as jnp
from jax import lax
from jax.experimental import pallas as pl
from jax.experimental.pallas import tpu as pltpu
from jax.experimental.pallas import tpu_sc as plsc

N = 320000
D = 128
NI = 10000
NIP = 10240
NC = 2
NS = 16
NW = NC * NS
RPW = N // NW
CH = 80
NCH = RPW // CH
IPT = NIP // NS
L = 16
f32 = jnp.float32
i32 = jnp.int32


def _k1_body(emb, labels3d, psum, pcnt, sums_acc,
             lblb, rowsb, cntloc, zb):
    c = lax.axis_index("c")
    s = lax.axis_index("s")
    wid = s * NC + c
    ones = jnp.full((L,), 1.0, f32)

    def init_zb(i, carry):
        for k in range(D // L):
            zb[i, pl.ds(k * L, L)] = jnp.zeros((L,), f32)
        return carry

    lax.fori_loop(0, 80, init_zb, 0)

    def init_zc(i, carry):
        cntloc[pl.ds(i * L, L)] = jnp.zeros((L,), f32)
        return carry

    lax.fori_loop(0, NIP // L, init_zc, 0)

    for k in range(IPT // 80):
        pltpu.sync_copy(zb, sums_acc.at[pl.ds(s * IPT + k * 80, 80)])
    plsc.subcore_barrier()

    pltpu.sync_copy(labels3d.at[wid], lblb)

    def chunk(j, carry):
        pltpu.sync_copy(emb.at[pl.ds(wid * RPW + j * CH, CH)], rowsb)
        pltpu.sync_copy(rowsb, sums_acc.at[lblb.at[j]], add=True)
        for k in range(CH // L):
            lv = lblb[j, pl.ds(k * L, L)]
            plsc.addupdate_scatter(cntloc, [lv], ones)
        return carry

    lax.fori_loop(0, NCH, chunk, 0)

    pltpu.sync_copy(cntloc, pcnt.at[wid])
    plsc.subcore_barrier()
    pltpu.sync_copy(sums_acc.at[pl.ds(s * IPT, IPT)],
                    psum.at[c, pl.ds(s * IPT, IPT)])


NR = NIP // 128


def _mid_body(pcnt_ref, rank_ref, dst_ref, scale_ref):
    cnt = jnp.sum(pcnt_ref[...], axis=0)
    presf = jnp.where(cnt > 0.0, 1.0, 0.0).astype(f32)
    col = lax.broadcasted_iota(i32, (128, 128), 0)
    row = lax.broadcasted_iota(i32, (128, 128), 1)
    tri_incl = jnp.where(col <= row, 1.0, 0.0).astype(f32)
    incl_row = jax.lax.dot(presf, tri_incl,
                           precision=jax.lax.Precision.HIGHEST)
    row_tot = incl_row[:, 127:128]
    colr = lax.broadcasted_iota(i32, (NR, NR), 0)
    rowr = lax.broadcasted_iota(i32, (NR, NR), 1)
    tri_excl = jnp.where(rowr < colr, 1.0, 0.0).astype(f32)
    off = jax.lax.dot(tri_excl, row_tot,
                      precision=jax.lax.Precision.HIGHEST)
    incl = incl_row + off
    rank = incl - presf
    u_tot = jnp.sum(presf)
    vmat = (lax.broadcasted_iota(i32, (NR, 128), 0) * 128
            + lax.broadcasted_iota(i32, (NR, 128), 1)).astype(f32)
    dstf = jnp.where(presf > 0.0, rank, u_tot + (vmat - rank))
    rank_ref[...] = rank.astype(i32)
    dst_ref[...] = dstf.astype(i32)
    scale_ref[...] = presf / jnp.maximum(cnt, 1.0)


def _k2_body(psum, dst3, scale2, aggs, scaleb, dstb, p0b, p1b, valb):
    t = lax.axis_index("s")
    base = t * IPT

    pltpu.sync_copy(dst3.at[t], dstb)
    pltpu.sync_copy(scale2.at[t], scaleb.at[pl.ds(0, IPT)])

    for cc in range(IPT // 128):
        rbase = base + cc * 128
        pltpu.sync_copy(psum.at[0, pl.ds(rbase, 128)], p0b)
        pltpu.sync_copy(psum.at[1, pl.ds(rbase, 128)], p1b)

        def rowfn(i, carry, cc=cc):
            sc = scaleb[pl.ds(cc * 128 + i, L)][0]
            for k in range(D // L):
                valb[i, pl.ds(k * L, L)] = (
                    p0b[i, pl.ds(k * L, L)] + p1b[i, pl.ds(k * L, L)]) * sc
            return carry

        lax.fori_loop(0, 128, rowfn, 0)
        pltpu.sync_copy(valb, aggs.at[dstb.at[cc]])


def _lower_bound(rkb, target):
    lo = jnp.zeros((L,), i32)
    bit = 8192
    while bit:
        idx = lo + (bit - 1)
        idxc = jnp.minimum(idx, NI - 1)
        rv = plsc.load_gather(rkb, [idxc])
        take = jnp.logical_and(idx < NI, rv < target)
        lo = lo + jnp.where(take, bit, 0)
        bit //= 2
    return lo


def _k3_body(aggs, rank_hbm, labels, agg_out, uniq_out, inv_out,
             aggb, uqb, rkb, lbl0, lblf, invb):
    c = lax.axis_index("c")
    s = lax.axis_index("s")
    wid = s * NC + c
    iota = lax.iota(i32, L)

    pltpu.sync_copy(rank_hbm.at[pl.ds(0, NI + L)], rkb)
    pltpu.sync_copy(labels.at[pl.ds(0, L)], lbl0)
    u_tot = rkb[pl.ds(NI, L)][0]
    minlab = lbl0[...][0]

    def uniq_block(b, wid_):
        jvec = iota + (wid_ * 320 + b * L)
        v = _lower_bound(rkb, jvec + 1) - 1
        uqb[pl.ds(b * L, L)] = jnp.where(jvec < u_tot, v, minlab)

    @pl.when(wid < NW - 1)
    def _():
        pltpu.sync_copy(aggs.at[pl.ds(wid * 320, 320)], aggb)
        pltpu.sync_copy(aggb, agg_out.at[pl.ds(wid * 320, 320)])
        for b in range(320 // L):
            uniq_block(b, wid)
        pltpu.sync_copy(uqb, uniq_out.at[pl.ds(wid * 320, 320)])

    @pl.when(wid == NW - 1)
    def _():
        pltpu.sync_copy(aggs.at[pl.ds(NI - 80, 80)], aggb.at[pl.ds(0, 80)])
        pltpu.sync_copy(aggb.at[pl.ds(0, 80)], agg_out.at[pl.ds(NI - 80, 80)])
        for b in range(80 // L):
            uniq_block(b, wid)
        pltpu.sync_copy(uqb.at[pl.ds(0, 80)], uniq_out.at[pl.ds(NI - 80, 80)])

    pltpu.sync_copy(labels.at[pl.ds(wid * RPW, RPW)], lblf)

    def blk(j, carry):
        lv = lblf[pl.ds(j * L, L)]
        invb[pl.ds(j * L, L)] = plsc.load_gather(rkb, [lv])
        return carry

    lax.fori_loop(0, RPW // L, blk, 0)
    pltpu.sync_copy(invb, inv_out.at[pl.ds(wid * RPW, RPW)])


def _sds(shape, dtype):
    return jax.ShapeDtypeStruct(shape, dtype)


def kernel(emb, labels):
    labels = labels.astype(i32)
    labels3d = labels.reshape(NW, NCH, CH)

    mesh = plsc.VectorSubcoreMesh(core_axis_name="c", subcore_axis_name="s")
    mesh1 = plsc.VectorSubcoreMesh(core_axis_name="c", subcore_axis_name="s",
                                   num_cores=1)

    k1 = pl.kernel(
        _k1_body,
        out_type=(_sds((NC, NIP, D), f32), _sds((NW, NIP), f32)),
        mesh=mesh,
        compiler_params=pltpu.CompilerParams(needs_layout_passes=False),
        scratch_types=[
            pltpu.VMEM_SHARED((NIP, D), f32),
            pltpu.VMEM((NCH, CH), i32),
            pltpu.VMEM((CH, D), f32),
            pltpu.VMEM((NIP,), f32),
            pltpu.VMEM((80, D), f32),
        ],
    )
    psum, pcnt = k1(emb, labels3d)

    mid = pl.pallas_call(
        _mid_body,
        out_shape=(_sds((NR, 128), i32), _sds((NR, 128), i32),
                   _sds((NR, 128), f32)),
    )
    rank2, dst2, scale2 = mid(pcnt.reshape(NW, NR, 128))
    rank_hbm = rank2.reshape(NIP)
    dst3 = dst2.reshape(NS, IPT // 128, 128)
    scale2d = scale2.reshape(NS, IPT)

    k2 = pl.kernel(
        _k2_body,
        out_type=_sds((NIP, D), f32),
        mesh=mesh1,
        compiler_params=pltpu.CompilerParams(needs_layout_passes=False),
        scratch_types=[
            pltpu.VMEM((IPT + L,), f32),
            pltpu.VMEM((IPT // 128, 128), i32),
            pltpu.VMEM((128, D), f32),
            pltpu.VMEM((128, D), f32),
            pltpu.VMEM((128, D), f32),
        ],
    )
    aggs = k2(psum, dst3, scale2d)

    k3 = pl.kernel(
        _k3_body,
        out_type=(_sds((NI, D), f32), _sds((NI,), i32), _sds((N,), i32)),
        mesh=mesh,
        compiler_params=pltpu.CompilerParams(needs_layout_passes=False),
        scratch_types=[
            pltpu.VMEM((320, D), f32),
            pltpu.VMEM((320,), i32),
            pltpu.VMEM((NI + L,), i32),
            pltpu.VMEM((L,), i32),
            pltpu.VMEM((RPW,), i32),
            pltpu.VMEM((RPW,), i32),
        ],
    )
    agg, uniq_ids, inv = k3(aggs, rank_hbm, labels)
    return (agg, uniq_ids, inv)


# speedup vs baseline: 12.9700x; 1.4386x over previous
import jax
import jax.numpy as jnp
from jax import lax
from jax.experimental import pallas as pl
from jax.experimental.pallas import tpu as pltpu
from jax.experimental.pallas import tpu_sc as plsc

N = 320000
D = 128
NI = 10000
NIP = 10240
NC = 2
NS = 16
NW = NC * NS
RPW = N // NW
CH = 80
NCH = RPW // CH
IPT = NIP // NS
L = 16
f32 = jnp.float32
i32 = jnp.int32


def _k1_body(emb, labels3d, psum, pcnt, sums_acc,
             lblb, rowsb, rows2, cntloc, sem0, sem1):
    c = lax.axis_index("c")
    s = lax.axis_index("s")
    wid = s * NC + c
    ones = jnp.full((L,), 1.0, f32)

    def init_zb(i, carry):
        for k in range(D // L):
            rowsb[i, pl.ds(k * L, L)] = jnp.zeros((L,), f32)
        return carry

    lax.fori_loop(0, CH, init_zb, 0)

    def init_zc(i, carry):
        cntloc[pl.ds(i * L, L)] = jnp.zeros((L,), f32)
        return carry

    lax.fori_loop(0, NIP // L, init_zc, 0)

    for k in range(IPT // CH):
        pltpu.sync_copy(rowsb, sums_acc.at[pl.ds(s * IPT + k * CH, CH)])
    plsc.subcore_barrier()

    pltpu.sync_copy(labels3d.at[wid], lblb)

    def cnt_chunk(j):
        for k in range(CH // L):
            lv = lblb[j, pl.ds(k * L, L)]
            plsc.addupdate_scatter(cntloc, [lv], ones)

    def ld(j, buf, sem):
        pltpu.async_copy(emb.at[pl.ds(wid * RPW + j * CH, CH)], buf, sem)

    def wt(j, buf, sem):
        pltpu.make_async_copy(
            emb.at[pl.ds(wid * RPW + j * CH, CH)], buf, sem).wait()

    ld(0, rowsb, sem0)

    def pair(g, carry):
        j0 = 2 * g
        ld(j0 + 1, rows2, sem1)
        wt(j0, rowsb, sem0)
        pltpu.sync_copy(rowsb, sums_acc.at[lblb.at[j0]], add=True)
        cnt_chunk(j0)
        ld(j0 + 2, rowsb, sem0)
        wt(j0 + 1, rows2, sem1)
        pltpu.sync_copy(rows2, sums_acc.at[lblb.at[j0 + 1]], add=True)
        cnt_chunk(j0 + 1)
        return carry

    lax.fori_loop(0, NCH // 2, pair, 0)
    wt(NCH - 1, rowsb, sem0)
    pltpu.sync_copy(rowsb, sums_acc.at[lblb.at[NCH - 1]], add=True)
    cnt_chunk(NCH - 1)

    pltpu.sync_copy(cntloc, pcnt.at[wid])
    plsc.subcore_barrier()
    pltpu.sync_copy(sums_acc.at[pl.ds(s * IPT, IPT)],
                    psum.at[c, pl.ds(s * IPT, IPT)])


NR = NIP // 128


def _mid_body(pcnt_ref, rank_ref, dst_ref, scale_ref):
    cnt = jnp.sum(pcnt_ref[...], axis=0)
    presf = jnp.where(cnt > 0.0, 1.0, 0.0).astype(f32)
    col = lax.broadcasted_iota(i32, (128, 128), 0)
    row = lax.broadcasted_iota(i32, (128, 128), 1)
    tri_incl = jnp.where(col <= row, 1.0, 0.0).astype(f32)
    incl_row = jax.lax.dot(presf, tri_incl,
                           precision=jax.lax.Precision.HIGHEST)
    row_tot = incl_row[:, 127:128]
    colr = lax.broadcasted_iota(i32, (NR, NR), 0)
    rowr = lax.broadcasted_iota(i32, (NR, NR), 1)
    tri_excl = jnp.where(rowr < colr, 1.0, 0.0).astype(f32)
    off = jax.lax.dot(tri_excl, row_tot,
                      precision=jax.lax.Precision.HIGHEST)
    incl = incl_row + off
    rank = incl - presf
    u_tot = jnp.sum(presf)
    vmat = (lax.broadcasted_iota(i32, (NR, 128), 0) * 128
            + lax.broadcasted_iota(i32, (NR, 128), 1)).astype(f32)
    dstf = jnp.where(presf > 0.0, rank, u_tot + (vmat - rank))
    rank_ref[...] = rank.astype(i32)
    dst_ref[...] = dstf.astype(i32)
    scale_ref[...] = presf / jnp.maximum(cnt, 1.0)


def _k2_body(psum, dst3, scale2, aggs, scaleb, dstb, p0b, p1b, valb):
    t = lax.axis_index("s")
    base = t * IPT

    pltpu.sync_copy(dst3.at[t], dstb)
    pltpu.sync_copy(scale2.at[t], scaleb.at[pl.ds(0, IPT)])

    for cc in range(IPT // 128):
        rbase = base + cc * 128
        pltpu.sync_copy(psum.at[0, pl.ds(rbase, 128)], p0b)
        pltpu.sync_copy(psum.at[1, pl.ds(rbase, 128)], p1b)

        def rowfn(i, carry, cc=cc):
            sc = scaleb[pl.ds(cc * 128 + i, L)][0]
            for k in range(D // L):
                valb[i, pl.ds(k * L, L)] = (
                    p0b[i, pl.ds(k * L, L)] + p1b[i, pl.ds(k * L, L)]) * sc
            return carry

        lax.fori_loop(0, 128, rowfn, 0)
        pltpu.sync_copy(valb, aggs.at[dstb.at[cc]])


def _lower_bound(rkb, target):
    lo = jnp.zeros((L,), i32)
    bit = 8192
    while bit:
        idx = lo + (bit - 1)
        idxc = jnp.minimum(idx, NI - 1)
        rv = plsc.load_gather(rkb, [idxc])
        take = jnp.logical_and(idx < NI, rv < target)
        lo = lo + jnp.where(take, bit, 0)
        bit //= 2
    return lo


def _k3_body(aggs, rank_hbm, labels, agg_out, uniq_out, inv_out,
             aggb, uqb, rkb, lbl0, lblf, invb):
    c = lax.axis_index("c")
    s = lax.axis_index("s")
    wid = s * NC + c
    iota = lax.iota(i32, L)

    pltpu.sync_copy(rank_hbm.at[pl.ds(0, NI + L)], rkb)
    pltpu.sync_copy(labels.at[pl.ds(0, L)], lbl0)
    u_tot = rkb[pl.ds(NI, L)][0]
    minlab = lbl0[...][0]

    def uniq_block(b, wid_):
        jvec = iota + (wid_ * 320 + b * L)
        v = _lower_bound(rkb, jvec + 1) - 1
        uqb[pl.ds(b * L, L)] = jnp.where(jvec < u_tot, v, minlab)

    @pl.when(wid < NW - 1)
    def _():
        pltpu.sync_copy(aggs.at[pl.ds(wid * 320, 320)], aggb)
        pltpu.sync_copy(aggb, agg_out.at[pl.ds(wid * 320, 320)])
        for b in range(320 // L):
            uniq_block(b, wid)
        pltpu.sync_copy(uqb, uniq_out.at[pl.ds(wid * 320, 320)])

    @pl.when(wid == NW - 1)
    def _():
        pltpu.sync_copy(aggs.at[pl.ds(NI - 80, 80)], aggb.at[pl.ds(0, 80)])
        pltpu.sync_copy(aggb.at[pl.ds(0, 80)], agg_out.at[pl.ds(NI - 80, 80)])
        for b in range(80 // L):
            uniq_block(b, wid)
        pltpu.sync_copy(uqb.at[pl.ds(0, 80)], uniq_out.at[pl.ds(NI - 80, 80)])

    pltpu.sync_copy(labels.at[pl.ds(wid * RPW, RPW)], lblf)

    def blk(j, carry):
        lv = lblf[pl.ds(j * L, L)]
        invb[pl.ds(j * L, L)] = plsc.load_gather(rkb, [lv])
        return carry

    lax.fori_loop(0, RPW // L, blk, 0)
    pltpu.sync_copy(invb, inv_out.at[pl.ds(wid * RPW, RPW)])


def _sds(shape, dtype):
    return jax.ShapeDtypeStruct(shape, dtype)


def kernel(emb, labels):
    labels = labels.astype(i32)
    labels3d = labels.reshape(NW, NCH, CH)

    mesh = plsc.VectorSubcoreMesh(core_axis_name="c", subcore_axis_name="s")
    mesh1 = plsc.VectorSubcoreMesh(core_axis_name="c", subcore_axis_name="s",
                                   num_cores=1)

    k1 = pl.kernel(
        _k1_body,
        out_type=(_sds((NC, NIP, D), f32), _sds((NW, NIP), f32)),
        mesh=mesh,
        compiler_params=pltpu.CompilerParams(needs_layout_passes=False),
        scratch_types=[
            pltpu.VMEM_SHARED((NIP, D), f32),
            pltpu.VMEM((NCH, CH), i32),
            pltpu.VMEM((CH, D), f32),
            pltpu.VMEM((CH, D), f32),
            pltpu.VMEM((NIP,), f32),
            pltpu.SemaphoreType.DMA,
            pltpu.SemaphoreType.DMA,
        ],
    )
    psum, pcnt = k1(emb, labels3d)

    mid = pl.pallas_call(
        _mid_body,
        out_shape=(_sds((NR, 128), i32), _sds((NR, 128), i32),
                   _sds((NR, 128), f32)),
    )
    rank2, dst2, scale2 = mid(pcnt.reshape(NW, NR, 128))
    rank_hbm = rank2.reshape(NIP)
    dst3 = dst2.reshape(NS, IPT // 128, 128)
    scale2d = scale2.reshape(NS, IPT)

    k2 = pl.kernel(
        _k2_body,
        out_type=_sds((NIP, D), f32),
        mesh=mesh1,
        compiler_params=pltpu.CompilerParams(needs_layout_passes=False),
        scratch_types=[
            pltpu.VMEM((IPT + L,), f32),
            pltpu.VMEM((IPT // 128, 128), i32),
            pltpu.VMEM((128, D), f32),
            pltpu.VMEM((128, D), f32),
            pltpu.VMEM((128, D), f32),
        ],
    )
    aggs = k2(psum, dst3, scale2d)

    k3 = pl.kernel(
        _k3_body,
        out_type=(_sds((NI, D), f32), _sds((NI,), i32), _sds((N,), i32)),
        mesh=mesh,
        compiler_params=pltpu.CompilerParams(needs_layout_passes=False),
        scratch_types=[
            pltpu.VMEM((320, D), f32),
            pltpu.VMEM((320,), i32),
            pltpu.VMEM((NI + L,), i32),
            pltpu.VMEM((L,), i32),
            pltpu.VMEM((RPW,), i32),
            pltpu.VMEM((RPW,), i32),
        ],
    )
    agg, uniq_ids, inv = k3(aggs, rank_hbm, labels)
    return (agg, uniq_ids, inv)


# speedup vs baseline: 13.9790x; 1.0778x over previous
import jax
import jax.numpy as jnp
from jax import lax
from jax.experimental import pallas as pl
from jax.experimental.pallas import tpu as pltpu
from jax.experimental.pallas import tpu_sc as plsc

N = 320000
D = 128
NI = 10000
NIP = 10240
NC = 2
NS = 16
NW = NC * NS
RPW = N // NW
CH = 80
NCH = RPW // CH
IPT = NIP // NS
L = 16
f32 = jnp.float32
i32 = jnp.int32


def _k1_body(emb, labels3d, psum, pcnt, sums_acc,
             lblb, rowsb, rows2, cntloc, sem0, sem1):
    c = lax.axis_index("c")
    s = lax.axis_index("s")
    wid = s * NC + c
    ones = jnp.full((L,), 1.0, f32)

    def init_zb(i, carry):
        for k in range(D // L):
            rowsb[i, pl.ds(k * L, L)] = jnp.zeros((L,), f32)
        return carry

    lax.fori_loop(0, CH, init_zb, 0)

    def init_zc(i, carry):
        cntloc[pl.ds(i * L, L)] = jnp.zeros((L,), f32)
        return carry

    lax.fori_loop(0, NIP // L, init_zc, 0)

    for k in range(IPT // CH):
        pltpu.sync_copy(rowsb, sums_acc.at[pl.ds(s * IPT + k * CH, CH)])
    plsc.subcore_barrier()

    pltpu.sync_copy(labels3d.at[wid], lblb)

    def cnt_chunk(j):
        for k in range(CH // L):
            lv = lblb[j, pl.ds(k * L, L)]
            plsc.addupdate_scatter(cntloc, [lv], ones)

    def ld(j, buf, sem):
        pltpu.async_copy(emb.at[pl.ds(wid * RPW + j * CH, CH)], buf, sem)

    def wt(j, buf, sem):
        pltpu.make_async_copy(
            emb.at[pl.ds(wid * RPW + j * CH, CH)], buf, sem).wait()

    ld(0, rowsb, sem0)

    def pair(g, carry):
        j0 = 2 * g
        ld(j0 + 1, rows2, sem1)
        wt(j0, rowsb, sem0)
        pltpu.sync_copy(rowsb, sums_acc.at[lblb.at[j0]], add=True)
        cnt_chunk(j0)
        ld(j0 + 2, rowsb, sem0)
        wt(j0 + 1, rows2, sem1)
        pltpu.sync_copy(rows2, sums_acc.at[lblb.at[j0 + 1]], add=True)
        cnt_chunk(j0 + 1)
        return carry

    lax.fori_loop(0, NCH // 2, pair, 0)
    wt(NCH - 1, rowsb, sem0)
    pltpu.sync_copy(rowsb, sums_acc.at[lblb.at[NCH - 1]], add=True)
    cnt_chunk(NCH - 1)

    pltpu.sync_copy(cntloc, pcnt.at[wid])
    plsc.subcore_barrier()
    pltpu.sync_copy(sums_acc.at[pl.ds(s * IPT, IPT)],
                    psum.at[c, pl.ds(s * IPT, IPT)])


NR = NIP // 128


def _mid_body(pcnt_ref, rank_ref, dst_ref, scale_ref):
    cnt = jnp.sum(pcnt_ref[...], axis=0)
    presf = jnp.where(cnt > 0.0, 1.0, 0.0).astype(f32)
    col = lax.broadcasted_iota(i32, (128, 128), 0)
    row = lax.broadcasted_iota(i32, (128, 128), 1)
    tri_incl = jnp.where(col <= row, 1.0, 0.0).astype(f32)
    incl_row = jax.lax.dot(presf, tri_incl,
                           precision=jax.lax.Precision.HIGHEST)
    row_tot = incl_row[:, 127:128]
    colr = lax.broadcasted_iota(i32, (NR, NR), 0)
    rowr = lax.broadcasted_iota(i32, (NR, NR), 1)
    tri_excl = jnp.where(rowr < colr, 1.0, 0.0).astype(f32)
    off = jax.lax.dot(tri_excl, row_tot,
                      precision=jax.lax.Precision.HIGHEST)
    incl = incl_row + off
    rank = incl - presf
    u_tot = jnp.sum(presf)
    vmat = (lax.broadcasted_iota(i32, (NR, 128), 0) * 128
            + lax.broadcasted_iota(i32, (NR, 128), 1)).astype(f32)
    dstf = jnp.where(presf > 0.0, rank, u_tot + (vmat - rank))
    rank_ref[...] = rank.astype(i32)
    dst_ref[...] = dstf.astype(i32)
    scale_ref[...] = presf / jnp.maximum(cnt, 1.0)


IPW = NIP // NW


def _lower_bound(rkb, target):
    lo = jnp.zeros((L,), i32)
    bit = 8192
    while bit:
        idx = lo + (bit - 1)
        idxc = jnp.minimum(idx, NI - 1)
        rv = plsc.load_gather(rkb, [idxc])
        take = jnp.logical_and(idx < NI, rv < target)
        lo = lo + jnp.where(take, bit, 0)
        bit //= 2
    return lo


def _k23_body(psum, dst3, scale2, rank_hbm, labels, aggs, uniq_out, inv_out,
              scaleb, dstb, p0b, p1b, valb, uqb, rkb, lbl0, lblf, invb):
    c = lax.axis_index("c")
    s = lax.axis_index("s")
    wid = s * NC + c
    iota = lax.iota(i32, L)

    pltpu.sync_copy(dst3.at[s], dstb)
    pltpu.sync_copy(scale2.at[s], scaleb.at[pl.ds(0, IPT)])

    def chunk_out(cc):
        rbase = s * IPT + cc * 128
        pltpu.sync_copy(psum.at[0, pl.ds(rbase, 128)], p0b)
        pltpu.sync_copy(psum.at[1, pl.ds(rbase, 128)], p1b)

        def rowfn(i, carry, cc=cc):
            sc = scaleb[pl.ds(cc * 128 + i, L)][0]
            for k in range(D // L):
                valb[i, pl.ds(k * L, L)] = (
                    p0b[i, pl.ds(k * L, L)] + p1b[i, pl.ds(k * L, L)]) * sc
            return carry

        lax.fori_loop(0, 128, rowfn, 0)
        pltpu.sync_copy(valb, aggs.at[dstb.at[cc]])

    @pl.when(c == 0)
    def _():
        for cc in (0, 1, 2):
            chunk_out(cc)

    @pl.when(c == 1)
    def _():
        for cc in (3, 4):
            chunk_out(cc)

    pltpu.sync_copy(rank_hbm.at[pl.ds(0, NI + L)], rkb)
    pltpu.sync_copy(labels.at[pl.ds(0, L)], lbl0)
    u_tot = rkb[pl.ds(NI, L)][0]
    minlab = lbl0[...][0]

    def uniq_block(b, wid_):
        jvec = iota + (wid_ * IPW + b * L)
        v = _lower_bound(rkb, jvec + 1) - 1
        uqb[pl.ds(b * L, L)] = jnp.where(jvec < u_tot, v, minlab)

    @pl.when(wid < NW - 1)
    def _():
        for b in range(IPW // L):
            uniq_block(b, wid)
        pltpu.sync_copy(uqb, uniq_out.at[pl.ds(wid * IPW, IPW)])

    @pl.when(wid == NW - 1)
    def _():
        for b in range(80 // L):
            uniq_block(b, wid)
        pltpu.sync_copy(uqb.at[pl.ds(0, 80)], uniq_out.at[pl.ds(NI - 80, 80)])

    pltpu.sync_copy(labels.at[pl.ds(wid * RPW, RPW)], lblf)

    def blk(j, carry):
        lv = lblf[pl.ds(j * L, L)]
        invb[pl.ds(j * L, L)] = plsc.load_gather(rkb, [lv])
        return carry

    lax.fori_loop(0, RPW // L, blk, 0)
    pltpu.sync_copy(invb, inv_out.at[pl.ds(wid * RPW, RPW)])


def _sds(shape, dtype):
    return jax.ShapeDtypeStruct(shape, dtype)


def kernel(emb, labels):
    labels = labels.astype(i32)
    labels3d = labels.reshape(NW, NCH, CH)

    mesh = plsc.VectorSubcoreMesh(core_axis_name="c", subcore_axis_name="s")

    k1 = pl.kernel(
        _k1_body,
        out_type=(_sds((NC, NIP, D), f32), _sds((NW, NIP), f32)),
        mesh=mesh,
        compiler_params=pltpu.CompilerParams(needs_layout_passes=False),
        scratch_types=[
            pltpu.VMEM_SHARED((NIP, D), f32),
            pltpu.VMEM((NCH, CH), i32),
            pltpu.VMEM((CH, D), f32),
            pltpu.VMEM((CH, D), f32),
            pltpu.VMEM((NIP,), f32),
            pltpu.SemaphoreType.DMA,
            pltpu.SemaphoreType.DMA,
        ],
    )
    psum, pcnt = k1(emb, labels3d)

    mid = pl.pallas_call(
        _mid_body,
        out_shape=(_sds((NR, 128), i32), _sds((NR, 128), i32),
                   _sds((NR, 128), f32)),
    )
    rank2, dst2, scale2 = mid(pcnt.reshape(NW, NR, 128))
    rank_hbm = rank2.reshape(NIP)
    dst3 = dst2.reshape(NS, IPT // 128, 128)
    scale2d = scale2.reshape(NS, IPT)

    k23 = pl.kernel(
        _k23_body,
        out_type=(_sds((NIP, D), f32), _sds((NI,), i32), _sds((N,), i32)),
        mesh=mesh,
        compiler_params=pltpu.CompilerParams(needs_layout_passes=False),
        scratch_types=[
            pltpu.VMEM((IPT + L,), f32),
            pltpu.VMEM((IPT // 128, 128), i32),
            pltpu.VMEM((128, D), f32),
            pltpu.VMEM((128, D), f32),
            pltpu.VMEM((128, D), f32),
            pltpu.VMEM((IPW,), i32),
            pltpu.VMEM((NI + L,), i32),
            pltpu.VMEM((L,), i32),
            pltpu.VMEM((RPW,), i32),
            pltpu.VMEM((RPW,), i32),
        ],
    )
    aggs, uniq_ids, inv = k23(psum, dst3, scale2d, rank_hbm, labels)
    return (aggs[:NI], uniq_ids, inv)
